# baseline (speedup 1.0000x reference)
import functools

import jax
import jax.numpy as jnp
from jax import lax
from jax.experimental import pallas as pl
from jax.experimental.pallas import tpu as pltpu
from jax.experimental.pallas import tpu_sc as plsc

N = 10000
E = 320000
D = 128
D2 = D // 2

NC = 2
NS = 16
L = 16
NT = NC * NS

ET = E // NT
CHUNK = 64
CPT = 160
CPT0 = 88
CPT1 = 2 * CPT - CPT0
PPC = 2
EP = NT * CPT * CHUNK
NA = 10112
NB = 4
SHIFT = 14

_mesh = plsc.VectorSubcoreMesh(core_axis_name="c", subcore_axis_name="s")


@functools.partial(
    pl.kernel,
    out_type=jax.ShapeDtypeStruct((NT, N), jnp.float32),
    mesh=_mesh,
    scratch_types=[
        pltpu.VMEM((ET,), jnp.int32),
        pltpu.VMEM((N,), jnp.float32),
    ],
    compiler_params=pltpu.CompilerParams(needs_layout_passes=False),
)
def _deg_kernel(dst_hbm, out_hbm, idx_v, hist_v):
    c = lax.axis_index("c")
    s = lax.axis_index("s")
    wid = c * NS + s
    pltpu.sync_copy(dst_hbm.at[wid], idx_v)

    def zero_body(i, carry):
        hist_v[pl.ds(i * L, L)] = jnp.zeros((L,), jnp.float32)
        return carry

    lax.fori_loop(0, N // L, zero_body, 0)

    ones = jnp.ones((L,), jnp.float32)

    def body(i, carry):
        idx = idx_v[pl.ds(i * L, L)]
        plsc.addupdate_scatter(hist_v, [idx], ones)
        return carry

    lax.fori_loop(0, ET // L, body, 0)
    pltpu.sync_copy(hist_v, out_hbm.at[wid])


def _prep_body(parts_ref, x_ref, xsh_ref, dinv_ref, wself_ref):
    parts = parts_ref[...]
    ones = jnp.ones((NT, 1), jnp.float32)
    deg = lax.dot_general(parts, ones, (((0,), (0,)), ((), ())),
                          preferred_element_type=jnp.float32) + 1.0
    dinv = lax.rsqrt(deg)
    xs = dinv * x_ref[...]
    xsh_ref[0, :, :] = xs
    xsh_ref[1, :, :] = xs
    dinv_ref[...] = dinv
    wself_ref[...] = 1.0 / deg


_prep_kernel = pl.pallas_call(
    _prep_body,
    out_shape=[
        jax.ShapeDtypeStruct((NC, N, D), jnp.float32),
        jax.ShapeDtypeStruct((N, 1), jnp.float32),
        jax.ShapeDtypeStruct((N, 1), jnp.float32),
    ],
)


@functools.partial(
    pl.kernel,
    out_type=jax.ShapeDtypeStruct((NC, NA, D), jnp.float32),
    mesh=_mesh,
    scratch_types=[
        pltpu.VMEM((CPT1 // PPC, 128), jnp.int32),
        pltpu.VMEM((NB, 2, CHUNK), jnp.int32),
        pltpu.VMEM((NB * CHUNK, D), jnp.float32),
        pltpu.VMEM_SHARED((NA, D), jnp.float32),
        pltpu.SemaphoreType.DMA((NB,)),
        pltpu.SemaphoreType.DMA((NB,)),
    ],
    compiler_params=pltpu.CompilerParams(use_tc_tiling_on_sc=False),
)
def _agg_kernel(pk_hbm, xsh_hbm, out_hbm, idxp, ring, rows, agg, sem_g, sem_s):
    c = lax.axis_index("c")
    s = lax.axis_index("s")
    wid = c * NS + s

    def zb(t, carry):
        r = t // (D // L)
        l = t % (D // L)
        rows[r, pl.ds(l * L, L)] = jnp.zeros((L,), jnp.float32)
        return carry

    lax.fori_loop(0, CHUNK * (D // L), zb, 0)

    zpt = NA // NS

    def za(k, carry):
        pltpu.sync_copy(rows.at[pl.ds(0, CHUNK)],
                        agg.at[pl.ds(s * zpt + k * CHUNK, CHUNK)])
        return carry

    lax.fori_loop(0, zpt // CHUNK, za, 0)
    pltpu.sync_copy(rows.at[pl.ds(0, zpt % CHUNK)],
                    agg.at[pl.ds(s * zpt + (zpt // CHUNK) * CHUNK, zpt % CHUNK)])
    plsc.subcore_barrier()

    pltpu.sync_copy(pk_hbm.at[wid], idxp)
    xs_hbm = xsh_hbm.at[c]

    mask = jnp.full((L,), (1 << SHIFT) - 1, jnp.int32)
    shift = jnp.full((L,), SHIFT, jnp.int32)

    def unpack(j, jmod, b):
        for q in range(CHUNK // L):
            p = idxp[j // PPC, pl.ds(jmod * CHUNK + q * L, L)]
            ring[b, 0, pl.ds(q * L, L)] = lax.bitwise_and(p, mask)
            ring[b, 1, pl.ds(q * L, L)] = lax.shift_right_logical(p, shift)

    def wait_gather(b):
        pltpu.make_async_copy(xs_hbm.at[ring.at[b, 0]],
                              rows.at[pl.ds(b * CHUNK, CHUNK)],
                              sem_g.at[b]).wait()

    def issue_gather(b):
        pltpu.async_copy(xs_hbm.at[ring.at[b, 0]],
                         rows.at[pl.ds(b * CHUNK, CHUNK)], sem_g.at[b])

    def issue_scatter(b):
        pltpu.async_copy(rows.at[pl.ds(b * CHUNK, CHUNK)],
                         agg.at[ring.at[b, 1]], sem_s.at[b], add=True)

    def wait_scatter(b):
        pltpu.make_async_copy(rows.at[pl.ds(b * CHUNK, CHUNK)],
                              agg.at[ring.at[b, 1]], sem_s.at[b]).wait()

    for b in range(NB - 1):
        unpack(b, b % PPC, b)
        issue_gather(b)
    wait_gather(0)
    issue_scatter(0)
    unpack(NB - 1, (NB - 1) % PPC, NB - 1)
    issue_gather(NB - 1)

    def body(g, carry):
        for i in range(NB):
            j = g * NB + i + 1
            b = (i + 1) % NB
            b1 = i
            wait_gather(b)
            issue_scatter(b)
            wait_scatter(b1)
            unpack(j + NB - 1, i % PPC, b1)
            issue_gather(b1)
        return carry

    n_groups = jnp.where(c == 0, (CPT0 - NB) // NB, (CPT1 - NB) // NB)
    lax.fori_loop(0, n_groups, body, 0)

    for i in range(1, NB):
        b = i % NB
        wait_gather(b)
        issue_scatter(b)
        wait_scatter((i - 1) % NB)
    wait_scatter((NB - 1) % NB)
    plsc.subcore_barrier()

    ro = NA // NS
    pltpu.sync_copy(agg.at[pl.ds(s * ro, ro)], out_hbm.at[c].at[pl.ds(s * ro, ro)])


def _out_body(a0_ref, a1_ref, x_ref, dinv_ref, wself_ref,
              wm_ref, bm_ref, wl_ref, bl_ref, mu_ref, ls_ref):
    h = dinv_ref[...] * (a0_ref[...] + a1_ref[...]) + wself_ref[...] * x_ref[...]
    mu_ref[...] = jnp.dot(h, wm_ref[...], preferred_element_type=jnp.float32) + bm_ref[...]
    ls_ref[...] = jnp.dot(h, wl_ref[...], preferred_element_type=jnp.float32) + bl_ref[...]


_R2 = 1000


_out_kernel = pl.pallas_call(
    _out_body,
    grid=(N // _R2,),
    in_specs=[
        pl.BlockSpec((_R2, D), lambda i: (i, 0)),
        pl.BlockSpec((_R2, D), lambda i: (i, 0)),
        pl.BlockSpec((_R2, D), lambda i: (i, 0)),
        pl.BlockSpec((_R2, 1), lambda i: (i, 0)),
        pl.BlockSpec((_R2, 1), lambda i: (i, 0)),
        pl.BlockSpec((D, D), lambda i: (0, 0)),
        pl.BlockSpec((1, D), lambda i: (0, 0)),
        pl.BlockSpec((D, D), lambda i: (0, 0)),
        pl.BlockSpec((1, D), lambda i: (0, 0)),
    ],
    out_specs=[
        pl.BlockSpec((_R2, D), lambda i: (i, 0)),
        pl.BlockSpec((_R2, D), lambda i: (i, 0)),
    ],
    out_shape=[
        jax.ShapeDtypeStruct((N, D), jnp.float32),
        jax.ShapeDtypeStruct((N, D), jnp.float32),
    ],
)


@jax.jit
def kernel(x, edge_index, W_mu, b_mu, W_logstd, b_logstd):
    src = edge_index[0]
    dst = edge_index[1]

    deg_parts = _deg_kernel(dst.reshape(NT, ET))
    xsh, dinv, wself = _prep_kernel(deg_parts, x)

    pad = EP - E
    flat = jnp.concatenate([
        src | (dst << SHIFT),
        jnp.full((pad,), N << SHIFT, jnp.int32),
    ])
    e0 = NS * CPT0 * CHUNK
    part0 = flat[:e0].reshape(NS, CPT0 * CHUNK)
    part0 = jnp.pad(part0, ((0, 0), (0, (CPT1 - CPT0) * CHUNK)),
                    constant_values=N << SHIFT)
    part1 = flat[e0:].reshape(NS, CPT1 * CHUNK)
    packed = jnp.concatenate([part0, part1]).reshape(NT, CPT1 // PPC, 128)
    agg = _agg_kernel(packed, xsh)

    mu, logstd = _out_kernel(agg[0], agg[1], x, dinv, wself,
                             W_mu, b_mu.reshape(1, D), W_logstd, b_logstd.reshape(1, D))
    return (mu, logstd)


# speedup vs baseline: 1.0459x; 1.0459x over previous
import functools

import jax
import jax.numpy as jnp
from jax import lax
from jax.experimental import pallas as pl
from jax.experimental.pallas import tpu as pltpu
from jax.experimental.pallas import tpu_sc as plsc

N = 10000
E = 320000
D = 128
D2 = D // 2

NC = 2
NS = 16
L = 16
NT = NC * NS

ET = E // NT
CHUNK = 64
CPT = 160
CPT0 = 232
CPT1 = 2 * CPT - CPT0
PPC = 2
EP = NT * CPT * CHUNK
NA = 10112
NB = 4
SHIFT = 14

_mesh = plsc.VectorSubcoreMesh(core_axis_name="c", subcore_axis_name="s")


@functools.partial(
    pl.kernel,
    out_type=jax.ShapeDtypeStruct((NT, N), jnp.float32),
    mesh=_mesh,
    scratch_types=[
        pltpu.VMEM((ET,), jnp.int32),
        pltpu.VMEM((N,), jnp.float32),
    ],
    compiler_params=pltpu.CompilerParams(needs_layout_passes=False),
)
def _deg_kernel(dst_hbm, out_hbm, idx_v, hist_v):
    c = lax.axis_index("c")
    s = lax.axis_index("s")
    wid = c * NS + s
    pltpu.sync_copy(dst_hbm.at[wid], idx_v)

    def zero_body(i, carry):
        hist_v[pl.ds(i * L, L)] = jnp.zeros((L,), jnp.float32)
        return carry

    lax.fori_loop(0, N // L, zero_body, 0)

    ones = jnp.ones((L,), jnp.float32)

    def body(i, carry):
        idx = idx_v[pl.ds(i * L, L)]
        plsc.addupdate_scatter(hist_v, [idx], ones)
        return carry

    lax.fori_loop(0, ET // L, body, 0)
    pltpu.sync_copy(hist_v, out_hbm.at[wid])


def _prep_body(parts_ref, x_ref, xsh_ref, dinv_ref, wself_ref):
    parts = parts_ref[...]
    ones = jnp.ones((NT, 1), jnp.float32)
    deg = lax.dot_general(parts, ones, (((0,), (0,)), ((), ())),
                          preferred_element_type=jnp.float32) + 1.0
    dinv = lax.rsqrt(deg)
    xs = dinv * x_ref[...]
    xsh_ref[0, :, :] = xs
    xsh_ref[1, :, :] = xs
    dinv_ref[...] = dinv
    wself_ref[...] = 1.0 / deg


_prep_kernel = pl.pallas_call(
    _prep_body,
    out_shape=[
        jax.ShapeDtypeStruct((NC, N, D), jnp.float32),
        jax.ShapeDtypeStruct((N, 1), jnp.float32),
        jax.ShapeDtypeStruct((N, 1), jnp.float32),
    ],
)


@functools.partial(
    pl.kernel,
    out_type=jax.ShapeDtypeStruct((NC, NA, D), jnp.float32),
    mesh=_mesh,
    scratch_types=[
        pltpu.VMEM((max(CPT0, CPT1) // PPC, 128), jnp.int32),
        pltpu.VMEM((NB, 2, CHUNK), jnp.int32),
        pltpu.VMEM((NB * CHUNK, D), jnp.float32),
        pltpu.VMEM_SHARED((NA, D), jnp.float32),
        pltpu.SemaphoreType.DMA((NB,)),
        pltpu.SemaphoreType.DMA((NB,)),
    ],
    compiler_params=pltpu.CompilerParams(use_tc_tiling_on_sc=False),
)
def _agg_kernel(pk_hbm, xsh_hbm, out_hbm, idxp, ring, rows, agg, sem_g, sem_s):
    c = lax.axis_index("c")
    s = lax.axis_index("s")
    wid = c * NS + s

    def zb(t, carry):
        r = t // (D // L)
        l = t % (D // L)
        rows[r, pl.ds(l * L, L)] = jnp.zeros((L,), jnp.float32)
        return carry

    lax.fori_loop(0, CHUNK * (D // L), zb, 0)

    zpt = NA // NS

    def za(k, carry):
        pltpu.sync_copy(rows.at[pl.ds(0, CHUNK)],
                        agg.at[pl.ds(s * zpt + k * CHUNK, CHUNK)])
        return carry

    lax.fori_loop(0, zpt // CHUNK, za, 0)
    pltpu.sync_copy(rows.at[pl.ds(0, zpt % CHUNK)],
                    agg.at[pl.ds(s * zpt + (zpt // CHUNK) * CHUNK, zpt % CHUNK)])
    plsc.subcore_barrier()

    pltpu.sync_copy(pk_hbm.at[wid], idxp)
    xs_hbm = xsh_hbm.at[c]

    mask = jnp.full((L,), (1 << SHIFT) - 1, jnp.int32)
    shift = jnp.full((L,), SHIFT, jnp.int32)

    def unpack(j, jmod, b):
        for q in range(CHUNK // L):
            p = idxp[j // PPC, pl.ds(jmod * CHUNK + q * L, L)]
            ring[b, 0, pl.ds(q * L, L)] = lax.bitwise_and(p, mask)
            ring[b, 1, pl.ds(q * L, L)] = lax.shift_right_logical(p, shift)

    def wait_gather(b):
        pltpu.make_async_copy(xs_hbm.at[ring.at[b, 0]],
                              rows.at[pl.ds(b * CHUNK, CHUNK)],
                              sem_g.at[b]).wait()

    def issue_gather(b):
        pltpu.async_copy(xs_hbm.at[ring.at[b, 0]],
                         rows.at[pl.ds(b * CHUNK, CHUNK)], sem_g.at[b])

    def issue_scatter(b):
        pltpu.async_copy(rows.at[pl.ds(b * CHUNK, CHUNK)],
                         agg.at[ring.at[b, 1]], sem_s.at[b], add=True)

    def wait_scatter(b):
        pltpu.make_async_copy(rows.at[pl.ds(b * CHUNK, CHUNK)],
                              agg.at[ring.at[b, 1]], sem_s.at[b]).wait()

    for b in range(NB - 1):
        unpack(b, b % PPC, b)
        issue_gather(b)
    wait_gather(0)
    issue_scatter(0)
    unpack(NB - 1, (NB - 1) % PPC, NB - 1)
    issue_gather(NB - 1)

    def body(g, carry):
        for i in range(NB):
            j = g * NB + i + 1
            b = (i + 1) % NB
            b1 = i
            wait_gather(b)
            issue_scatter(b)
            wait_scatter(b1)
            unpack(j + NB - 1, i % PPC, b1)
            issue_gather(b1)
        return carry

    n_groups = jnp.where(c == 0, (CPT0 - NB) // NB, (CPT1 - NB) // NB)
    lax.fori_loop(0, n_groups, body, 0)

    for i in range(1, NB):
        b = i % NB
        wait_gather(b)
        issue_scatter(b)
        wait_scatter((i - 1) % NB)
    wait_scatter((NB - 1) % NB)
    plsc.subcore_barrier()

    ro = NA // NS
    pltpu.sync_copy(agg.at[pl.ds(s * ro, ro)], out_hbm.at[c].at[pl.ds(s * ro, ro)])


def _out_body(a0_ref, a1_ref, x_ref, dinv_ref, wself_ref,
              wm_ref, bm_ref, wl_ref, bl_ref, mu_ref, ls_ref):
    h = dinv_ref[...] * (a0_ref[...] + a1_ref[...]) + wself_ref[...] * x_ref[...]
    mu_ref[...] = jnp.dot(h, wm_ref[...], preferred_element_type=jnp.float32) + bm_ref[...]
    ls_ref[...] = jnp.dot(h, wl_ref[...], preferred_element_type=jnp.float32) + bl_ref[...]


_R2 = 1000


_out_kernel = pl.pallas_call(
    _out_body,
    grid=(N // _R2,),
    in_specs=[
        pl.BlockSpec((_R2, D), lambda i: (i, 0)),
        pl.BlockSpec((_R2, D), lambda i: (i, 0)),
        pl.BlockSpec((_R2, D), lambda i: (i, 0)),
        pl.BlockSpec((_R2, 1), lambda i: (i, 0)),
        pl.BlockSpec((_R2, 1), lambda i: (i, 0)),
        pl.BlockSpec((D, D), lambda i: (0, 0)),
        pl.BlockSpec((1, D), lambda i: (0, 0)),
        pl.BlockSpec((D, D), lambda i: (0, 0)),
        pl.BlockSpec((1, D), lambda i: (0, 0)),
    ],
    out_specs=[
        pl.BlockSpec((_R2, D), lambda i: (i, 0)),
        pl.BlockSpec((_R2, D), lambda i: (i, 0)),
    ],
    out_shape=[
        jax.ShapeDtypeStruct((N, D), jnp.float32),
        jax.ShapeDtypeStruct((N, D), jnp.float32),
    ],
)


@jax.jit
def kernel(x, edge_index, W_mu, b_mu, W_logstd, b_logstd):
    src = edge_index[0]
    dst = edge_index[1]

    deg_parts = _deg_kernel(dst.reshape(NT, ET))
    xsh, dinv, wself = _prep_kernel(deg_parts, x)

    pad = EP - E
    flat = jnp.concatenate([
        src | (dst << SHIFT),
        jnp.full((pad,), N << SHIFT, jnp.int32),
    ])
    cmax = max(CPT0, CPT1)
    e0 = NS * CPT0 * CHUNK
    part0 = flat[:e0].reshape(NS, CPT0 * CHUNK)
    part0 = jnp.pad(part0, ((0, 0), (0, (cmax - CPT0) * CHUNK)),
                    constant_values=N << SHIFT)
    part1 = flat[e0:].reshape(NS, CPT1 * CHUNK)
    part1 = jnp.pad(part1, ((0, 0), (0, (cmax - CPT1) * CHUNK)),
                    constant_values=N << SHIFT)
    packed = jnp.concatenate([part0, part1]).reshape(NT, cmax // PPC, 128)
    agg = _agg_kernel(packed, xsh)

    mu, logstd = _out_kernel(agg[0], agg[1], x, dinv, wself,
                             W_mu, b_mu.reshape(1, D), W_logstd, b_logstd.reshape(1, D))
    return (mu, logstd)


# speedup vs baseline: 2.3556x; 2.2521x over previous
import functools

import jax
import jax.numpy as jnp
from jax import lax
from jax.experimental import pallas as pl
from jax.experimental.pallas import tpu as pltpu
from jax.experimental.pallas import tpu_sc as plsc

N = 10000
E = 320000
D = 128
D2 = D // 2

NC = 2
NS = 16
L = 16
NT = NC * NS

ET = E // NT
CHUNK = 64
CPT = 320
PPC = 2
EP = NS * CPT * CHUNK
NA = 10112
NB = 4
SHIFT = 14

_mesh = plsc.VectorSubcoreMesh(core_axis_name="c", subcore_axis_name="s")


@functools.partial(
    pl.kernel,
    out_type=jax.ShapeDtypeStruct((NT, N), jnp.float32),
    mesh=_mesh,
    scratch_types=[
        pltpu.VMEM((ET,), jnp.int32),
        pltpu.VMEM((N,), jnp.float32),
    ],
    compiler_params=pltpu.CompilerParams(needs_layout_passes=False),
)
def _deg_kernel(dst_hbm, out_hbm, idx_v, hist_v):
    c = lax.axis_index("c")
    s = lax.axis_index("s")
    wid = c * NS + s
    pltpu.sync_copy(dst_hbm.at[wid], idx_v)

    def zero_body(i, carry):
        hist_v[pl.ds(i * L, L)] = jnp.zeros((L,), jnp.float32)
        return carry

    lax.fori_loop(0, N // L, zero_body, 0)

    ones = jnp.ones((L,), jnp.float32)

    def body(i, carry):
        idx = idx_v[pl.ds(i * L, L)]
        plsc.addupdate_scatter(hist_v, [idx], ones)
        return carry

    lax.fori_loop(0, ET // L, body, 0)
    pltpu.sync_copy(hist_v, out_hbm.at[wid])


def _prep_body(parts_ref, x_ref, xsh_ref, dinv_ref, wself_ref):
    parts = parts_ref[...]
    ones = jnp.ones((NT, 1), jnp.float32)
    deg = lax.dot_general(parts, ones, (((0,), (0,)), ((), ())),
                          preferred_element_type=jnp.float32) + 1.0
    dinv = lax.rsqrt(deg)
    xs = dinv * x_ref[...]
    xsh_ref[0, :N, :] = xs[:, :D2]
    xsh_ref[1, :N, :] = xs[:, D2:]
    dinv_ref[...] = dinv
    wself_ref[...] = 1.0 / deg


_prep_kernel = pl.pallas_call(
    _prep_body,
    out_shape=[
        jax.ShapeDtypeStruct((NC, NA, D2), jnp.float32),
        jax.ShapeDtypeStruct((N, 1), jnp.float32),
        jax.ShapeDtypeStruct((N, 1), jnp.float32),
    ],
)


@functools.partial(
    pl.kernel,
    out_type=jax.ShapeDtypeStruct((NC, NA, D2), jnp.float32),
    mesh=_mesh,
    scratch_types=[
        pltpu.VMEM((CPT // PPC, 128), jnp.int32),
        pltpu.VMEM((NB, 2, CHUNK), jnp.int32),
        pltpu.VMEM((NB * CHUNK, D2), jnp.float32),
        pltpu.VMEM_SHARED((NA, D2), jnp.float32),
        pltpu.VMEM_SHARED((NA, D2), jnp.float32),
        pltpu.SemaphoreType.DMA((NB,)),
        pltpu.SemaphoreType.DMA((NB,)),
    ],
    compiler_params=pltpu.CompilerParams(use_tc_tiling_on_sc=False),
)
def _agg_kernel(pk_hbm, xsh_hbm, out_hbm, idxp, ring, rows, agg, xs_sp,
                sem_g, sem_s):
    c = lax.axis_index("c")
    s = lax.axis_index("s")

    def zb(t, carry):
        r = t // (D2 // L)
        l = t % (D2 // L)
        rows[r, pl.ds(l * L, L)] = jnp.zeros((L,), jnp.float32)
        return carry

    lax.fori_loop(0, CHUNK * (D2 // L), zb, 0)

    zpt = NA // NS

    def za(k, carry):
        pltpu.sync_copy(rows.at[pl.ds(0, CHUNK)],
                        agg.at[pl.ds(s * zpt + k * CHUNK, CHUNK)])
        return carry

    lax.fori_loop(0, zpt // CHUNK, za, 0)
    pltpu.sync_copy(rows.at[pl.ds(0, zpt % CHUNK)],
                    agg.at[pl.ds(s * zpt + (zpt // CHUNK) * CHUNK, zpt % CHUNK)])
    pltpu.sync_copy(xsh_hbm.at[c].at[pl.ds(s * zpt, zpt)],
                    xs_sp.at[pl.ds(s * zpt, zpt)])
    plsc.subcore_barrier()

    pltpu.sync_copy(pk_hbm.at[s], idxp)
    xs_hbm = xs_sp

    mask = jnp.full((L,), (1 << SHIFT) - 1, jnp.int32)
    shift = jnp.full((L,), SHIFT, jnp.int32)

    def unpack(j, jmod, b):
        for q in range(CHUNK // L):
            p = idxp[j // PPC, pl.ds(jmod * CHUNK + q * L, L)]
            ring[b, 0, pl.ds(q * L, L)] = lax.bitwise_and(p, mask)
            ring[b, 1, pl.ds(q * L, L)] = lax.shift_right_logical(p, shift)

    def wait_gather(b):
        pltpu.make_async_copy(xs_hbm.at[ring.at[b, 0]],
                              rows.at[pl.ds(b * CHUNK, CHUNK)],
                              sem_g.at[b]).wait()

    def issue_gather(b):
        pltpu.async_copy(xs_hbm.at[ring.at[b, 0]],
                         rows.at[pl.ds(b * CHUNK, CHUNK)], sem_g.at[b])

    def issue_scatter(b):
        pltpu.async_copy(rows.at[pl.ds(b * CHUNK, CHUNK)],
                         agg.at[ring.at[b, 1]], sem_s.at[b], add=True)

    def wait_scatter(b):
        pltpu.make_async_copy(rows.at[pl.ds(b * CHUNK, CHUNK)],
                              agg.at[ring.at[b, 1]], sem_s.at[b]).wait()

    for b in range(NB - 1):
        unpack(b, b % PPC, b)
        issue_gather(b)
    wait_gather(0)
    issue_scatter(0)
    unpack(NB - 1, (NB - 1) % PPC, NB - 1)
    issue_gather(NB - 1)

    def body(g, carry):
        for i in range(NB):
            j = g * NB + i + 1
            b = (i + 1) % NB
            b1 = i
            wait_gather(b)
            issue_scatter(b)
            wait_scatter(b1)
            unpack(j + NB - 1, i % PPC, b1)
            issue_gather(b1)
        return carry

    lax.fori_loop(0, (CPT - NB) // NB, body, 0)

    for j in range(CPT - NB + 1, CPT):
        b = j % NB
        wait_gather(b)
        issue_scatter(b)
        wait_scatter((j - 1) % NB)
    wait_scatter((CPT - 1) % NB)
    plsc.subcore_barrier()

    ro = NA // NS
    pltpu.sync_copy(agg.at[pl.ds(s * ro, ro)], out_hbm.at[c].at[pl.ds(s * ro, ro)])


def _out_body(a0_ref, a1_ref, x_ref, dinv_ref, wself_ref,
              wm_ref, bm_ref, wl_ref, bl_ref, mu_ref, ls_ref):
    a = jnp.concatenate([a0_ref[...], a1_ref[...]], axis=1)
    h = dinv_ref[...] * a + wself_ref[...] * x_ref[...]
    mu_ref[...] = jnp.dot(h, wm_ref[...], preferred_element_type=jnp.float32) + bm_ref[...]
    ls_ref[...] = jnp.dot(h, wl_ref[...], preferred_element_type=jnp.float32) + bl_ref[...]


_R2 = 1000


_out_kernel = pl.pallas_call(
    _out_body,
    grid=(N // _R2,),
    in_specs=[
        pl.BlockSpec((_R2, D2), lambda i: (i, 0)),
        pl.BlockSpec((_R2, D2), lambda i: (i, 0)),
        pl.BlockSpec((_R2, D), lambda i: (i, 0)),
        pl.BlockSpec((_R2, 1), lambda i: (i, 0)),
        pl.BlockSpec((_R2, 1), lambda i: (i, 0)),
        pl.BlockSpec((D, D), lambda i: (0, 0)),
        pl.BlockSpec((1, D), lambda i: (0, 0)),
        pl.BlockSpec((D, D), lambda i: (0, 0)),
        pl.BlockSpec((1, D), lambda i: (0, 0)),
    ],
    out_specs=[
        pl.BlockSpec((_R2, D), lambda i: (i, 0)),
        pl.BlockSpec((_R2, D), lambda i: (i, 0)),
    ],
    out_shape=[
        jax.ShapeDtypeStruct((N, D), jnp.float32),
        jax.ShapeDtypeStruct((N, D), jnp.float32),
    ],
)


@jax.jit
def kernel(x, edge_index, W_mu, b_mu, W_logstd, b_logstd):
    src = edge_index[0]
    dst = edge_index[1]

    deg_parts = _deg_kernel(dst.reshape(NT, ET))
    xsh, dinv, wself = _prep_kernel(deg_parts, x)

    pad = EP - E
    packed = jnp.concatenate([
        src | (dst << SHIFT),
        jnp.full((pad,), N << SHIFT, jnp.int32),
    ]).reshape(NS, CPT // PPC, 128)
    agg = _agg_kernel(packed, xsh)

    mu, logstd = _out_kernel(agg[0], agg[1], x, dinv, wself,
                             W_mu, b_mu.reshape(1, D), W_logstd, b_logstd.reshape(1, D))
    return (mu, logstd)
